# baseline (speedup 1.0000x reference)
import functools

import jax
import jax.numpy as jnp
from jax import lax
from jax.experimental import pallas as pl
from jax.experimental.pallas import tpu as pltpu
from jax.experimental.pallas import tpu_sc as plsc

N = 100000
F = 128
S = 64
NC = 2
NS = 16
L = 16
CHUNK = 128
NFULL = N // CHUNK
REM = N - NFULL * CHUNK
FH = F // NC
KV = FH // L
GROUP = 6
GROWS = GROUP * CHUNK
NGROUPS = 8
NEXTRA = NFULL - NS * NGROUPS * GROUP

_mesh = plsc.VectorSubcoreMesh(core_axis_name="c", subcore_axis_name="s")


@functools.partial(
    pl.kernel,
    out_type=jax.ShapeDtypeStruct((S, F), jnp.float32),
    mesh=_mesh,
    scratch_types=[
        pltpu.VMEM((GROWS, FH), jnp.float32),
        pltpu.VMEM((GROWS, FH), jnp.float32),
        pltpu.VMEM((GROWS,), jnp.int32),
        pltpu.VMEM((GROWS,), jnp.int32),
        pltpu.VMEM((CHUNK // 2,), jnp.int32),
        pltpu.VMEM((CHUNK // 2,), jnp.int32),
        pltpu.VMEM((CHUNK, FH), jnp.float32),
        pltpu.VMEM((CHUNK,), jnp.int32),
        pltpu.VMEM((REM, FH), jnp.float32),
        pltpu.VMEM((REM,), jnp.int32),
        pltpu.VMEM((S, FH), jnp.float32),
        pltpu.VMEM((S,), jnp.int32),
        pltpu.VMEM_SHARED((S, FH), jnp.float32),
        pltpu.SemaphoreType.DMA,
        pltpu.SemaphoreType.DMA,
        pltpu.SemaphoreType.DMA,
        pltpu.SemaphoreType.DMA,
    ],
    compiler_params=pltpu.CompilerParams(
        use_tc_tiling_on_sc=False, needs_layout_passes=False),
)
def _seg_sum(x_hbm, b_hbm, out_hbm, xb0, xb1, ib0, ib1, is0, is1, xbe, ibe,
             xbr, ibr, acc, iref, acc_sh, sem0, sem1, sem_m, sem_s):
    core = lax.axis_index("c")
    sub = lax.axis_index("s")
    col0 = core * FH
    c0 = sub * (NGROUPS * GROUP) + jnp.minimum(sub, NEXTRA)
    r0 = c0 * CHUNK

    offs = [lax.iota(jnp.int32, L) + k * L for k in range(KV)]

    def start_group(g, xb, ib, isb, sem):
        pltpu.async_copy(
            x_hbm.at[pl.ds(r0 + g * GROWS, GROWS), pl.ds(col0, FH)], xb, sem)
        pltpu.async_copy(b_hbm.at[pl.ds(r0 + g * GROWS, GROWS)], ib, sem)
        pltpu.async_copy(b_hbm.at[pl.ds(r0 + g * GROWS, CHUNK // 2)], isb, sem)

    def wait_group(xb, ib, isb, sem):
        pltpu.make_async_copy(
            x_hbm.at[pl.ds(0, GROWS), pl.ds(col0, FH)], xb, sem).wait()
        pltpu.make_async_copy(b_hbm.at[pl.ds(0, GROWS)], ib, sem).wait()
        pltpu.make_async_copy(b_hbm.at[pl.ds(0, CHUNK // 2)], isb, sem).wait()

    def init_state():
        z = jnp.zeros((L,), jnp.float32)
        return (jnp.zeros((L,), jnp.int32), z, z, z, z)

    def accum_unit(xb, ib, u, st):
        idxv = ib[pl.ds(u * L, L)]
        nb = jnp.sum((idxv != st[0]).astype(jnp.int32))

        def fast(st2):
            regs = list(st2[1:])
            row0 = u * L
            for k in range(KV):
                vs = [xb[row0 + r, pl.ds(k * L, L)] for r in range(L)]
                while len(vs) > 1:
                    vs = [vs[i] + vs[i + 1] for i in range(0, len(vs), 2)]
                regs[k] = regs[k] + vs[0]
            return (st2[0], *regs)

        def slow(st2):
            prev = st2[0]
            regs = list(st2[1:])
            for r in range(L):
                seg = idxv[jnp.full((L,), r, jnp.int32)]
                mask = seg != prev
                row = u * L + r
                for k in range(KV):
                    plsc.addupdate_scatter(acc, [prev, offs[k]], regs[k],
                                           mask=mask)
                    v = xb[row, pl.ds(k * L, L)]
                    regs[k] = jnp.where(mask, v, regs[k] + v)
                prev = seg
            return (prev, *regs)

        return lax.cond(nb == 0, fast, slow, st)

    def flush(st):
        for k in range(KV):
            plsc.addupdate_scatter(acc, [st[0], offs[k]], st[1 + k])

    start_group(0, xb0, ib0, is0, sem0)
    start_group(1, xb1, ib1, is1, sem1)

    zeros16 = jnp.zeros((L,), jnp.float32)

    def zrow(r, carry):
        for k in range(KV):
            acc[r, pl.ds(k * L, L)] = zeros16
        return carry

    lax.fori_loop(0, S, zrow, 0)

    @pl.when(sub == 0)
    def _():
        pltpu.sync_copy(acc, acc_sh)

    for k in range(S // L):
        iref[pl.ds(k * L, L)] = lax.iota(jnp.int32, L) + k * L

    plsc.subcore_barrier()

    def body(i, st):
        for b, (xb, ib, isb, sem) in enumerate(
                ((xb0, ib0, is0, sem0), (xb1, ib1, is1, sem1))):
            g = 2 * i + b
            wait_group(xb, ib, isb, sem)

            d_sc = pltpu.async_copy(
                xb.at[pl.ds(0, CHUNK // 2)], acc_sh.at[isb], sem_s, add=True)

            def units(u, st2):
                return accum_unit(xb, ib, u, st2)

            st = lax.fori_loop(CHUNK // 2 // L, GROWS // L, units, st)
            d_sc.wait()

            @pl.when(g + 2 < NGROUPS)
            def _():
                start_group(g + 2, xb, ib, isb, sem)

        return st

    st = lax.fori_loop(0, NGROUPS // 2, body, init_state())
    flush(st)

    @pl.when(sub < NEXTRA)
    def _():
        re = r0 + NGROUPS * GROWS
        pltpu.sync_copy(x_hbm.at[pl.ds(re, CHUNK), pl.ds(col0, FH)], xbe)
        pltpu.sync_copy(b_hbm.at[pl.ds(re, CHUNK)], ibe)

        def units(u, st2):
            return accum_unit(xbe, ibe, u, st2)

        flush(lax.fori_loop(0, CHUNK // L, units, init_state()))

    @pl.when(sub == NS - 1)
    def _():
        rr = NFULL * CHUNK
        pltpu.sync_copy(x_hbm.at[pl.ds(rr, REM), pl.ds(col0, FH)], xbr)
        pltpu.sync_copy(b_hbm.at[pl.ds(rr, REM)], ibr)

        def units(u, st2):
            return accum_unit(xbr, ibr, u, st2)

        flush(lax.fori_loop(0, REM // L, units, init_state()))

    plsc.subcore_barrier()

    pltpu.async_copy(acc, acc_sh.at[iref], sem_m, add=True).wait()

    plsc.subcore_barrier()

    @pl.when(sub == 0)
    def _():
        pltpu.sync_copy(acc_sh, out_hbm.at[pl.ds(0, S), pl.ds(col0, FH)])


def kernel(x, batch, batch_size):
    del batch_size
    return _seg_sum(x, batch.astype(jnp.int32))


# speedup vs baseline: 1.0424x; 1.0424x over previous
import functools

import jax
import jax.numpy as jnp
from jax import lax
from jax.experimental import pallas as pl
from jax.experimental.pallas import tpu as pltpu
from jax.experimental.pallas import tpu_sc as plsc

N = 100000
F = 128
S = 64
NC = 2
NS = 16
L = 16
CHUNK = 128
NFULL = N // CHUNK
REM = N - NFULL * CHUNK
FH = F // NC
KV = FH // L
GROUP = 4
GROWS = GROUP * CHUNK
NGROUPS = 12
NEXTRA = NFULL - NS * NGROUPS * GROUP

_mesh = plsc.VectorSubcoreMesh(core_axis_name="c", subcore_axis_name="s")


@functools.partial(
    pl.kernel,
    out_type=jax.ShapeDtypeStruct((S, F), jnp.float32),
    mesh=_mesh,
    scratch_types=[
        pltpu.VMEM((GROWS, FH), jnp.float32),
        pltpu.VMEM((GROWS, FH), jnp.float32),
        pltpu.VMEM((GROWS,), jnp.int32),
        pltpu.VMEM((GROWS,), jnp.int32),
        pltpu.VMEM((CHUNK // 2,), jnp.int32),
        pltpu.VMEM((CHUNK // 2,), jnp.int32),
        pltpu.VMEM((CHUNK, FH), jnp.float32),
        pltpu.VMEM((CHUNK,), jnp.int32),
        pltpu.VMEM((REM, FH), jnp.float32),
        pltpu.VMEM((REM,), jnp.int32),
        pltpu.VMEM((S, FH), jnp.float32),
        pltpu.VMEM((S,), jnp.int32),
        pltpu.VMEM_SHARED((S, FH), jnp.float32),
        pltpu.SemaphoreType.DMA,
        pltpu.SemaphoreType.DMA,
        pltpu.SemaphoreType.DMA,
        pltpu.SemaphoreType.DMA,
    ],
    compiler_params=pltpu.CompilerParams(
        use_tc_tiling_on_sc=False, needs_layout_passes=False),
)
def _seg_sum(x_hbm, b_hbm, out_hbm, xb0, xb1, ib0, ib1, is0, is1, xbe, ibe,
             xbr, ibr, acc, iref, acc_sh, sem0, sem1, sem_m, sem_s):
    core = lax.axis_index("c")
    sub = lax.axis_index("s")
    col0 = core * FH
    c0 = sub * (NGROUPS * GROUP) + jnp.minimum(sub, NEXTRA)
    r0 = c0 * CHUNK

    offs = [lax.iota(jnp.int32, L) + k * L for k in range(KV)]

    def start_group(g, xb, ib, isb, sem):
        pltpu.async_copy(
            x_hbm.at[pl.ds(r0 + g * GROWS, GROWS), pl.ds(col0, FH)], xb, sem)
        pltpu.async_copy(b_hbm.at[pl.ds(r0 + g * GROWS, GROWS)], ib, sem)
        pltpu.async_copy(b_hbm.at[pl.ds(r0 + g * GROWS, CHUNK // 2)], isb, sem)

    def wait_group(xb, ib, isb, sem):
        pltpu.make_async_copy(
            x_hbm.at[pl.ds(0, GROWS), pl.ds(col0, FH)], xb, sem).wait()
        pltpu.make_async_copy(b_hbm.at[pl.ds(0, GROWS)], ib, sem).wait()
        pltpu.make_async_copy(b_hbm.at[pl.ds(0, CHUNK // 2)], isb, sem).wait()

    def init_state():
        z = jnp.zeros((L,), jnp.float32)
        return (jnp.zeros((L,), jnp.int32), z, z, z, z)

    def accum_unit(xb, ib, u, st):
        idxv = ib[pl.ds(u * L, L)]
        nb = jnp.sum((idxv != st[0]).astype(jnp.int32))

        def fast(st2):
            regs = list(st2[1:])
            row0 = u * L
            for k in range(KV):
                vs = [xb[row0 + r, pl.ds(k * L, L)] for r in range(L)]
                while len(vs) > 1:
                    vs = [vs[i] + vs[i + 1] for i in range(0, len(vs), 2)]
                regs[k] = regs[k] + vs[0]
            return (st2[0], *regs)

        def slow(st2):
            prev = st2[0]
            regs = list(st2[1:])
            for r in range(L):
                seg = idxv[jnp.full((L,), r, jnp.int32)]
                mask = seg != prev
                row = u * L + r
                for k in range(KV):
                    plsc.addupdate_scatter(acc, [prev, offs[k]], regs[k],
                                           mask=mask)
                    v = xb[row, pl.ds(k * L, L)]
                    regs[k] = jnp.where(mask, v, regs[k] + v)
                prev = seg
            return (prev, *regs)

        return lax.cond(nb == 0, fast, slow, st)

    def flush(st):
        for k in range(KV):
            plsc.addupdate_scatter(acc, [st[0], offs[k]], st[1 + k])

    start_group(0, xb0, ib0, is0, sem0)
    start_group(1, xb1, ib1, is1, sem1)

    zeros16 = jnp.zeros((L,), jnp.float32)

    def zrow(r, carry):
        for k in range(KV):
            acc[r, pl.ds(k * L, L)] = zeros16
        return carry

    lax.fori_loop(0, S, zrow, 0)

    @pl.when(sub == 0)
    def _():
        pltpu.sync_copy(acc, acc_sh)

    for k in range(S // L):
        iref[pl.ds(k * L, L)] = lax.iota(jnp.int32, L) + k * L

    plsc.subcore_barrier()

    def body(i, st):
        for b, (xb, ib, isb, sem) in enumerate(
                ((xb0, ib0, is0, sem0), (xb1, ib1, is1, sem1))):
            g = 2 * i + b
            wait_group(xb, ib, isb, sem)

            d_sc = pltpu.async_copy(
                xb.at[pl.ds(0, CHUNK // 2)], acc_sh.at[isb], sem_s, add=True)

            def units(u, st2):
                return accum_unit(xb, ib, u, st2)

            st = lax.fori_loop(CHUNK // 2 // L, GROWS // L, units, st)
            d_sc.wait()

            @pl.when(g + 2 < NGROUPS)
            def _():
                start_group(g + 2, xb, ib, isb, sem)

        return st

    st = lax.fori_loop(0, NGROUPS // 2, body, init_state())
    flush(st)

    @pl.when(sub < NEXTRA)
    def _():
        re = r0 + NGROUPS * GROWS
        pltpu.sync_copy(x_hbm.at[pl.ds(re, CHUNK), pl.ds(col0, FH)], xbe)
        pltpu.sync_copy(b_hbm.at[pl.ds(re, CHUNK)], ibe)

        def units(u, st2):
            return accum_unit(xbe, ibe, u, st2)

        flush(lax.fori_loop(0, CHUNK // L, units, init_state()))

    @pl.when(sub == NS - 1)
    def _():
        rr = NFULL * CHUNK
        pltpu.sync_copy(x_hbm.at[pl.ds(rr, REM), pl.ds(col0, FH)], xbr)
        pltpu.sync_copy(b_hbm.at[pl.ds(rr, REM)], ibr)

        def units(u, st2):
            return accum_unit(xbr, ibr, u, st2)

        flush(lax.fori_loop(0, REM // L, units, init_state()))

    plsc.subcore_barrier()

    pltpu.async_copy(acc, acc_sh.at[iref], sem_m, add=True).wait()

    plsc.subcore_barrier()

    @pl.when(sub == 0)
    def _():
        pltpu.sync_copy(acc_sh, out_hbm.at[pl.ds(0, S), pl.ds(col0, FH)])


def kernel(x, batch, batch_size):
    del batch_size
    return _seg_sum(x, batch.astype(jnp.int32))


# speedup vs baseline: 1.0430x; 1.0006x over previous
import functools

import jax
import jax.numpy as jnp
from jax import lax
from jax.experimental import pallas as pl
from jax.experimental.pallas import tpu as pltpu
from jax.experimental.pallas import tpu_sc as plsc

N = 100000
F = 128
S = 64
NC = 2
NS = 16
L = 16
CHUNK = 128
NFULL = N // CHUNK
REM = N - NFULL * CHUNK
FH = F // NC
KV = FH // L
GROUP = 4
GROWS = GROUP * CHUNK
NGROUPS = 12
NEXTRA = NFULL - NS * NGROUPS * GROUP

_mesh = plsc.VectorSubcoreMesh(
    core_axis_name="c", subcore_axis_name="s", num_cores=NC, num_subcores=NS)


@functools.partial(
    pl.kernel,
    out_type=jax.ShapeDtypeStruct((S, F), jnp.float32),
    mesh=_mesh,
    scratch_types=[
        pltpu.VMEM((GROWS, FH), jnp.float32),
        pltpu.VMEM((GROWS, FH), jnp.float32),
        pltpu.VMEM((GROWS,), jnp.int32),
        pltpu.VMEM((GROWS,), jnp.int32),
        pltpu.VMEM((CHUNK // 2,), jnp.int32),
        pltpu.VMEM((CHUNK // 2,), jnp.int32),
        pltpu.VMEM((CHUNK, FH), jnp.float32),
        pltpu.VMEM((CHUNK,), jnp.int32),
        pltpu.VMEM((REM, FH), jnp.float32),
        pltpu.VMEM((REM,), jnp.int32),
        pltpu.VMEM((S, FH), jnp.float32),
        pltpu.VMEM((S,), jnp.int32),
        pltpu.VMEM_SHARED((S, FH), jnp.float32),
        pltpu.SemaphoreType.DMA,
        pltpu.SemaphoreType.DMA,
        pltpu.SemaphoreType.DMA,
        pltpu.SemaphoreType.DMA,
    ],
    compiler_params=pltpu.CompilerParams(
        use_tc_tiling_on_sc=False, needs_layout_passes=False),
)
def _seg_sum(x_hbm, b_hbm, out_hbm, xb0, xb1, ib0, ib1, is0, is1, xbe, ibe,
             xbr, ibr, acc, iref, acc_sh, sem0, sem1, sem_m, sem_s):
    core = lax.axis_index("c")
    sub = lax.axis_index("s")
    col0 = core * FH
    c0 = sub * (NGROUPS * GROUP) + jnp.minimum(sub, NEXTRA)
    r0 = c0 * CHUNK

    offs = [lax.iota(jnp.int32, L) + k * L for k in range(KV)]

    def start_group(g, xb, ib, isb, sem):
        pltpu.async_copy(
            x_hbm.at[pl.ds(r0 + g * GROWS, GROWS), pl.ds(col0, FH)], xb, sem)
        pltpu.async_copy(b_hbm.at[pl.ds(r0 + g * GROWS, GROWS)], ib, sem)
        pltpu.async_copy(b_hbm.at[pl.ds(r0 + g * GROWS, CHUNK // 2)], isb, sem)

    def wait_group(xb, ib, isb, sem):
        pltpu.make_async_copy(
            x_hbm.at[pl.ds(0, GROWS), pl.ds(col0, FH)], xb, sem).wait()
        pltpu.make_async_copy(b_hbm.at[pl.ds(0, GROWS)], ib, sem).wait()
        pltpu.make_async_copy(b_hbm.at[pl.ds(0, CHUNK // 2)], isb, sem).wait()

    def init_state():
        z = jnp.zeros((L,), jnp.float32)
        return (jnp.zeros((L,), jnp.int32), z, z, z, z)

    def accum_unit(xb, ib, u, st):
        idxv = ib[pl.ds(u * L, L)]
        nb = jnp.sum((idxv != st[0]).astype(jnp.int32))

        def fast(st2):
            regs = list(st2[1:])
            row0 = u * L
            for k in range(KV):
                vs = [xb[row0 + r, pl.ds(k * L, L)] for r in range(L)]
                while len(vs) > 1:
                    vs = [vs[i] + vs[i + 1] for i in range(0, len(vs), 2)]
                regs[k] = regs[k] + vs[0]
            return (st2[0], *regs)

        def slow(st2):
            prev = st2[0]
            regs = list(st2[1:])
            for r in range(L):
                seg = idxv[jnp.full((L,), r, jnp.int32)]
                mask = seg != prev
                row = u * L + r
                for k in range(KV):
                    plsc.addupdate_scatter(acc, [prev, offs[k]], regs[k],
                                           mask=mask)
                    v = xb[row, pl.ds(k * L, L)]
                    regs[k] = jnp.where(mask, v, regs[k] + v)
                prev = seg
            return (prev, *regs)

        return lax.cond(nb == 0, fast, slow, st)

    def flush(st):
        for k in range(KV):
            plsc.addupdate_scatter(acc, [st[0], offs[k]], st[1 + k])

    start_group(0, xb0, ib0, is0, sem0)
    start_group(1, xb1, ib1, is1, sem1)

    zeros16 = jnp.zeros((L,), jnp.float32)

    def zrow(r, carry):
        for k in range(KV):
            acc[r, pl.ds(k * L, L)] = zeros16
        return carry

    lax.fori_loop(0, S, zrow, 0)

    @pl.when(sub == 0)
    def _():
        pltpu.sync_copy(acc, acc_sh)

    for k in range(S // L):
        iref[pl.ds(k * L, L)] = lax.iota(jnp.int32, L) + k * L

    plsc.subcore_barrier()

    def body(i, st):
        for b, (xb, ib, isb, sem) in enumerate(
                ((xb0, ib0, is0, sem0), (xb1, ib1, is1, sem1))):
            g = 2 * i + b
            wait_group(xb, ib, isb, sem)

            d_sc = pltpu.async_copy(
                xb.at[pl.ds(0, CHUNK // 2)], acc_sh.at[isb], sem_s, add=True)

            def units(u, st2):
                return accum_unit(xb, ib, u, st2)

            st = lax.fori_loop(CHUNK // 2 // L, GROWS // L, units, st)
            d_sc.wait()

            @pl.when(g + 2 < NGROUPS)
            def _():
                start_group(g + 2, xb, ib, isb, sem)

        return st

    st = lax.fori_loop(0, NGROUPS // 2, body, init_state())
    flush(st)

    @pl.when(sub < NEXTRA)
    def _():
        re = r0 + NGROUPS * GROWS
        pltpu.sync_copy(x_hbm.at[pl.ds(re, CHUNK), pl.ds(col0, FH)], xbe)
        pltpu.sync_copy(b_hbm.at[pl.ds(re, CHUNK)], ibe)

        def units(u, st2):
            return accum_unit(xbe, ibe, u, st2)

        flush(lax.fori_loop(0, CHUNK // L, units, init_state()))

    @pl.when(sub == NS - 1)
    def _():
        rr = NFULL * CHUNK
        pltpu.sync_copy(x_hbm.at[pl.ds(rr, REM), pl.ds(col0, FH)], xbr)
        pltpu.sync_copy(b_hbm.at[pl.ds(rr, REM)], ibr)

        def units(u, st2):
            return accum_unit(xbr, ibr, u, st2)

        flush(lax.fori_loop(0, REM // L, units, init_state()))

    plsc.subcore_barrier()

    pltpu.async_copy(acc, acc_sh.at[iref], sem_m, add=True).wait()

    plsc.subcore_barrier()

    @pl.when(sub == 0)
    def _():
        pltpu.sync_copy(acc_sh, out_hbm.at[pl.ds(0, S), pl.ds(col0, FH)])


def kernel(x, batch, batch_size):
    del batch_size
    return _seg_sum(x, batch.astype(jnp.int32))
